# baseline (speedup 1.0000x reference)
import jax
import jax.numpy as jnp
from jax.experimental import pallas as pl

_BLK = 2048


def _node5_body(x_ref, cw_ref, cb_ref, wt_ref, b_ref, h1_ref, h2_ref, h3_ref,
                h5_ref, o_ref):
    def tr(k):
        t = x_ref[:, k:k + 1] * cw_ref[0:1, k:k + 1] + cb_ref[0:1, k:k + 1]
        return jnp.maximum(t, 0.0)

    y = b_ref[0:1, :]
    y = y + tr(80) * wt_ref[0:1, :]
    y = y + tr(83) * wt_ref[1:2, :]
    y = y + tr(86) * wt_ref[2:3, :]
    y = y + tr(89) * wt_ref[3:4, :]
    y = y + h5_ref[:, 60:61] * wt_ref[4:5, :]
    y = y + h1_ref[:, 1:2] * wt_ref[5:6, :]
    y = y + h2_ref[:, 6:7] * wt_ref[6:7, :]
    y = y + h3_ref[:, 11:12] * wt_ref[7:8, :]
    o_ref[:, :] = jnp.maximum(y, 0.0)


def kernel(x, conv_w, conv_b, W0, b0, W1, b1, W2, b2, W3, b3, W4, b4, W5, b5,
           h1, h2, h3, h4, h5):
    B = x.shape[0]
    cw2 = conv_w.reshape(1, 128)
    cb2 = conv_b.reshape(1, 128)
    w5t = W5.T
    b52 = b5.reshape(1, 64)
    return pl.pallas_call(
        _node5_body,
        grid=(B // _BLK,),
        in_specs=[
            pl.BlockSpec((_BLK, 128), lambda i: (i, 0)),
            pl.BlockSpec((1, 128), lambda i: (0, 0)),
            pl.BlockSpec((1, 128), lambda i: (0, 0)),
            pl.BlockSpec((8, 64), lambda i: (0, 0)),
            pl.BlockSpec((1, 64), lambda i: (0, 0)),
            pl.BlockSpec((_BLK, 64), lambda i: (i, 0)),
            pl.BlockSpec((_BLK, 64), lambda i: (i, 0)),
            pl.BlockSpec((_BLK, 64), lambda i: (i, 0)),
            pl.BlockSpec((_BLK, 64), lambda i: (i, 0)),
        ],
        out_specs=pl.BlockSpec((_BLK, 64), lambda i: (i, 0)),
        out_shape=jax.ShapeDtypeStruct((B, 64), jnp.float32),
    )(x, cw2, cb2, w5t, b52, h1, h2, h3, h5)
